# baseline (speedup 1.0000x reference)
import jax
import jax.numpy as jnp
from jax.experimental import pallas as pl
from jax.experimental.pallas import tpu as pltpu

N_NODES = 512
D_FEAT = 256
N_HIDDEN = 256
_GROUP = 8


def _dot(a, b):
    return jax.lax.dot_general(
        a, b, (((1,), (0,)), ((), ())),
        preferred_element_type=jnp.float32)


def _gcn_kernel(gidx_ref, *refs):
    G = _GROUP
    x_refs = refs[0:G]
    a_refs = refs[G:2 * G]
    mbd_ref = refs[2 * G]
    (Win_ref, bin_ref, W1_ref, b1_ref, W2_ref, b2_ref,
     W3_ref, b3_ref) = refs[2 * G + 1:2 * G + 9]
    out_ref = refs[2 * G + 9]

    bf = jnp.bfloat16
    x2 = jnp.concatenate([r[0] for r in x_refs], axis=0).astype(bf)
    a_bf = [r[0].astype(bf) for r in a_refs]
    h0 = jax.nn.relu(_dot(x2, Win_ref[...].astype(bf)) + bin_ref[...])
    h = h0
    for w_ref, b_ref, act in ((W1_ref, b1_ref, jax.nn.relu),
                              (W2_ref, b2_ref, jax.nn.relu),
                              (W3_ref, b3_ref, jnp.tanh)):
        hb = h.astype(bf)
        ts = [_dot(a_bf[j], hb[j * N_NODES:(j + 1) * N_NODES])
              for j in range(G)]
        t = jnp.concatenate(ts, axis=0).astype(bf)
        h = act(_dot(t, w_ref[...].astype(bf)) + b_ref[...])
    hf = h + h0

    m = mbd_ref[0]
    cnt = jnp.sum(m, axis=1, keepdims=True)
    acc = _dot(m.astype(bf), hf.astype(bf))
    out_ref[0] = acc / jnp.maximum(cnt, 1.0)


def kernel(graph, coverpoint_mask, batch_xs, batch_as, W_in, b_in,
           W1, b1, W2, b2, W3, b3):
    B = graph.shape[0]
    G = _GROUP
    g = graph.astype(jnp.int32)
    mask_f = coverpoint_mask.astype(jnp.float32).reshape(B // G, G, 1, N_NODES)
    eye = jnp.eye(G, dtype=jnp.float32).reshape(1, G, G, 1)
    mbd = (mask_f * eye).reshape(B // G, G, G * N_NODES)

    xa_specs = []
    for j in range(G):
        xa_specs.append(pl.BlockSpec(
            (1, N_NODES, D_FEAT),
            lambda b, gi, j=j: (0, 0, 0)))
    for j in range(G):
        xa_specs.append(pl.BlockSpec(
            (1, N_NODES, N_NODES),
            lambda b, gi, j=j: (0, 0, 0)))
    mbd_spec = pl.BlockSpec((1, G, G * N_NODES), lambda b, gi: (b, 0, 0))
    w_specs = []
    for shape in ((D_FEAT, N_HIDDEN), (1, N_HIDDEN)) * 4:
        w_specs.append(pl.BlockSpec(shape, lambda b, gi: (0, 0)))

    grid_spec = pltpu.PrefetchScalarGridSpec(
        num_scalar_prefetch=1,
        grid=(B // G,),
        in_specs=xa_specs + [mbd_spec] + w_specs,
        out_specs=pl.BlockSpec((1, G, N_HIDDEN), lambda b, gi: (b, 0, 0)),
    )

    xa_args = [batch_xs] * G + [batch_as] * G

    out = pl.pallas_call(
        _gcn_kernel,
        grid_spec=grid_spec,
        out_shape=jax.ShapeDtypeStruct((B // G, G, N_HIDDEN), jnp.float32),
    )(g, *xa_args, mbd,
      W_in, b_in.reshape(1, N_HIDDEN), W1, b1.reshape(1, N_HIDDEN),
      W2, b2.reshape(1, N_HIDDEN), W3, b3.reshape(1, N_HIDDEN))
    return out.reshape(B, N_HIDDEN)


# speedup vs baseline: 1.0476x; 1.0476x over previous
import jax
import jax.numpy as jnp
from jax.experimental import pallas as pl
from jax.experimental.pallas import tpu as pltpu

N_NODES = 512
D_FEAT = 256
N_HIDDEN = 256
_GROUP = 8


def _dot(a, b):
    return jax.lax.dot_general(
        a, b, (((1,), (0,)), ((), ())),
        preferred_element_type=jnp.float32)


def _gcn_kernel(gidx_ref, *refs):
    G = _GROUP
    x_refs = refs[0:G]
    a_refs = refs[G:2 * G]
    mbd_ref = refs[2 * G]
    (Win_ref, bin_ref, W1_ref, b1_ref, W2_ref, b2_ref,
     W3_ref, b3_ref) = refs[2 * G + 1:2 * G + 9]
    out_ref = refs[2 * G + 9]

    bf = jnp.bfloat16
    x2 = jnp.concatenate([r[0] for r in x_refs], axis=0).astype(bf)
    a_bf = [r[0].astype(bf) for r in a_refs]
    h0 = jax.nn.relu(_dot(x2, Win_ref[...].astype(bf)) + bin_ref[...])
    h = h0
    for w_ref, b_ref, act in ((W1_ref, b1_ref, jax.nn.relu),
                              (W2_ref, b2_ref, jax.nn.relu),
                              (W3_ref, b3_ref, jnp.tanh)):
        hb = h.astype(bf)
        ts = [_dot(a_bf[j], hb[j * N_NODES:(j + 1) * N_NODES])
              for j in range(G)]
        t = jnp.concatenate(ts, axis=0).astype(bf)
        h = act(_dot(t, w_ref[...].astype(bf)) + b_ref[...])
    hf = h + h0

    mrows = mbd_ref[0]
    cnt = jnp.sum(mrows, axis=1, keepdims=True)
    tiled = jnp.concatenate([mrows] * G, axis=1)
    seg = jax.lax.broadcasted_iota(jnp.int32, (G, G * N_NODES), 1) // N_NODES
    lane = jax.lax.broadcasted_iota(jnp.int32, (G, G * N_NODES), 0)
    m = jnp.where(seg == lane, tiled, 0.0)
    acc = _dot(m.astype(bf), hf.astype(bf))
    out_ref[0] = acc / jnp.maximum(cnt, 1.0)


def kernel(graph, coverpoint_mask, batch_xs, batch_as, W_in, b_in,
           W1, b1, W2, b2, W3, b3):
    B = graph.shape[0]
    G = _GROUP
    g = graph.astype(jnp.int32)
    mbd = coverpoint_mask.astype(jnp.float32).reshape(B // G, G, N_NODES)

    xa_specs = []
    for j in range(G):
        xa_specs.append(pl.BlockSpec(
            (1, N_NODES, D_FEAT),
            lambda b, gi, j=j: (gi[G * b + j], 0, 0)))
    for j in range(G):
        xa_specs.append(pl.BlockSpec(
            (1, N_NODES, N_NODES),
            lambda b, gi, j=j: (gi[G * b + j], 0, 0)))
    mbd_spec = pl.BlockSpec((1, G, N_NODES), lambda b, gi: (b, 0, 0))
    w_specs = []
    for shape in ((D_FEAT, N_HIDDEN), (1, N_HIDDEN)) * 4:
        w_specs.append(pl.BlockSpec(shape, lambda b, gi: (0, 0)))

    grid_spec = pltpu.PrefetchScalarGridSpec(
        num_scalar_prefetch=1,
        grid=(B // G,),
        in_specs=xa_specs + [mbd_spec] + w_specs,
        out_specs=pl.BlockSpec((1, G, N_HIDDEN), lambda b, gi: (b, 0, 0)),
    )

    xa_args = [batch_xs] * G + [batch_as] * G

    out = pl.pallas_call(
        _gcn_kernel,
        grid_spec=grid_spec,
        out_shape=jax.ShapeDtypeStruct((B // G, G, N_HIDDEN), jnp.float32),
    )(g, *xa_args, mbd,
      W_in, b_in.reshape(1, N_HIDDEN), W1, b1.reshape(1, N_HIDDEN),
      W2, b2.reshape(1, N_HIDDEN), W3, b3.reshape(1, N_HIDDEN))
    return out.reshape(B, N_HIDDEN)
